# baseline (speedup 1.0000x reference)
import functools

import jax
import jax.numpy as jnp
from jax import lax
from jax.experimental import pallas as pl
from jax.experimental.pallas import tpu as pltpu
from jax.experimental.pallas import tpu_sc as plsc

N = 10000
E = 320000
H = 128
HE = 64
RADIUS = 3

NC = 2
NS = 16
NW = NC * NS
EC = E // NW
KB = 80
NB = EC // KB
NPT = N // NS
VPR = H // 16

_SC_MESH = dict(core_axis_name="c", subcore_axis_name="s",
                num_cores=NC, num_subcores=NS)

BE = 512
GE = E // BE
BN = 2000
GN = N // BN

_f32 = jnp.float32


def _sc_pass0_body(cx_hbm, cy_hbm, cz_hbm, src_hbm, dst_hbm, zerosH_hbm,
                   ones_hbm, deg_out, s_out,
                   cx_v, cy_v, cz_v, src_v, dst_v, s_v, ones_v, deg_sh):
    cid = lax.axis_index("c")
    sid = lax.axis_index("s")
    wid = sid * NC + cid
    pltpu.sync_copy(cx_hbm, cx_v)
    pltpu.sync_copy(cy_hbm, cy_v)
    pltpu.sync_copy(cz_hbm, cz_v)
    pltpu.sync_copy(ones_hbm, ones_v)
    pltpu.sync_copy(zerosH_hbm, deg_sh.at[pl.ds(sid * NPT, NPT)])
    plsc.subcore_barrier()

    @pl.loop(0, NB)
    def _blk(b):
        pltpu.sync_copy(src_hbm.at[wid, b], src_v)
        pltpu.sync_copy(dst_hbm.at[wid, b], dst_v)
        for j in range(KB // 16):
            si = src_v[pl.ds(j * 16, 16)]
            di = dst_v[pl.ds(j * 16, 16)]
            acc = jnp.zeros((16,), _f32)
            for cv in (cx_v, cy_v, cz_v):
                d = plsc.load_gather(cv, [si]) - plsc.load_gather(cv, [di])
                acc = acc + d * d
            s_v[pl.ds(j * 16, 16)] = acc
        pltpu.sync_copy(s_v, s_out.at[wid, b])
        pltpu.sync_copy(ones_v, deg_sh.at[dst_v], add=True)

    plsc.subcore_barrier()
    pltpu.sync_copy(deg_sh.at[pl.ds(sid * NPT, NPT)],
                    deg_out.at[cid, sid])


@functools.cache
def _sc_pass0():
    return pl.kernel(
        _sc_pass0_body,
        out_type=(jax.ShapeDtypeStruct((NC, NS, NPT, H), _f32),
                  jax.ShapeDtypeStruct((NW, NB, KB), _f32)),
        mesh=plsc.VectorSubcoreMesh(**_SC_MESH),
        compiler_params=pltpu.CompilerParams(needs_layout_passes=False),
        scratch_types=[
            pltpu.VMEM((N,), _f32),
            pltpu.VMEM((N,), _f32),
            pltpu.VMEM((N,), _f32),
            pltpu.VMEM((KB,), jnp.int32),
            pltpu.VMEM((KB,), jnp.int32),
            pltpu.VMEM((KB,), _f32),
            pltpu.VMEM((KB, H), _f32),
            pltpu.VMEM_SHARED((N, H), _f32),
        ],
    )


@functools.cache
def _make_edge_pass(write_h, mode="add"):
    def body(lin_hbm, tab_hbm, src_hbm, dst_hbm, scale_hbm, zeros_hbm, *rest):
        if write_h:
            (agg_out, h_out, src_v, dst_v, dsts_v, scale_v, lin_v, gath_v,
             agg_sh, sem_idx, sem_lin, sem_g, sem_sc, sem_h) = rest
        else:
            (agg_out, src_v, dst_v, dsts_v, scale_v, lin_v, gath_v,
             agg_sh, sem_idx, sem_lin, sem_g, sem_sc, sem_h) = rest
        cid = lax.axis_index("c")
        sid = lax.axis_index("s")
        wid = sid * NC + cid
        pltpu.sync_copy(zeros_hbm, agg_sh.at[pl.ds(sid * NPT, NPT)])
        plsc.subcore_barrier()
        ebase = wid * EC

        def lin_slice(b):
            return lin_hbm.at[pl.ds(ebase + b * KB, KB)]

        def issue_idx(b, k):
            pltpu.async_copy(src_hbm.at[wid, b], src_v.at[k], sem_idx.at[k])
            pltpu.async_copy(dst_hbm.at[wid, b], dst_v.at[k], sem_idx.at[k])
            pltpu.async_copy(scale_hbm.at[wid, b], scale_v.at[k],
                             sem_idx.at[k])

        def wait_idx(b, k):
            pltpu.make_async_copy(src_hbm.at[wid, b], src_v.at[k],
                                  sem_idx.at[k]).wait()
            pltpu.make_async_copy(dst_hbm.at[wid, b], dst_v.at[k],
                                  sem_idx.at[k]).wait()
            pltpu.make_async_copy(scale_hbm.at[wid, b], scale_v.at[k],
                                  sem_idx.at[k]).wait()

        def issue_lin(b, k):
            pltpu.async_copy(lin_slice(b), lin_v.at[k], sem_lin.at[k])

        def issue_gather(k):
            pltpu.async_copy(tab_hbm.at[src_v.at[k]], gath_v.at[k],
                             sem_g.at[k])

        def wait_gather(k):
            pltpu.make_async_copy(tab_hbm.at[src_v.at[k]], gath_v.at[k],
                                  sem_g.at[k]).wait()

        def issue_scatter(k):
            if mode == "add":
                pltpu.async_copy(gath_v.at[k], agg_sh.at[dsts_v.at[k]],
                                 sem_sc.at[k], add=True)
            elif mode == "linear":
                pltpu.async_copy(gath_v.at[k], agg_sh.at[pl.ds(k * KB, KB)],
                                 sem_sc.at[k])

        def wait_scatter(k):
            if mode == "add":
                pltpu.make_async_copy(gath_v.at[k], agg_sh.at[dsts_v.at[k]],
                                      sem_sc.at[k]).wait()
            elif mode == "linear":
                pltpu.make_async_copy(gath_v.at[k],
                                      agg_sh.at[pl.ds(k * KB, KB)],
                                      sem_sc.at[k]).wait()

        def block(b, k, tail):
            wait_gather(k)
            for j in range(KB // 16):
                sl = pl.ds(j * 16, 16)
                dsts_v.at[k][sl] = dst_v.at[k][sl]
            pltpu.make_async_copy(lin_slice(b), lin_v.at[k],
                                  sem_lin.at[k]).wait()
            lin_k = lin_v.at[k]
            gath_k = gath_v.at[k]
            scale_k = scale_v.at[k]

            @pl.loop(0, KB, unroll=2)
            def _edge(e):
                sc = plsc.load_gather(scale_k,
                                      [jnp.full((16,), e, jnp.int32)])
                sls = [pl.ds(v * 16, 16) for v in range(VPR)]
                lins = [lin_k[e, sl] for sl in sls]
                gats = [gath_k[e, sl] for sl in sls]
                hs = [jnp.maximum(a + b, 0.0) for a, b in zip(lins, gats)]
                vals = [h * sc for h in hs]
                for v in range(VPR):
                    if write_h:
                        lin_k[e, sls[v]] = hs[v]
                    gath_k[e, sls[v]] = vals[v]

            if not tail:
                @pl.when(b + 2 < NB)
                def _():
                    issue_idx(b + 2, k)
                    if not write_h:
                        issue_lin(b + 2, k)
            if not tail:
                @pl.when(b >= 1)
                def _():
                    wait_scatter(k ^ 1)
                wait_idx(b + 1, k ^ 1)
                issue_gather(k ^ 1)
            issue_scatter(k)
            if write_h:
                pltpu.async_copy(lin_k, h_out.at[pl.ds(ebase + b * KB, KB)],
                                 sem_h.at[k])
                pltpu.make_async_copy(lin_k,
                                      h_out.at[pl.ds(ebase + b * KB, KB)],
                                      sem_h.at[k]).wait()
                if not tail:
                    @pl.when(b + 2 < NB)
                    def _():
                        issue_lin(b + 2, k)

        issue_idx(0, 0)
        issue_idx(1, 1)
        issue_lin(0, 0)
        issue_lin(1, 1)
        wait_idx(0, 0)
        issue_gather(0)

        @pl.loop(0, NB - 1, step=2)
        def _pair(b0):
            block(b0, 0, False)
            block(b0 + 1, 1, False)

        block(NB - 1, 0, True)
        wait_scatter(0)
        wait_scatter(1)

        plsc.subcore_barrier()
        pltpu.sync_copy(agg_sh.at[pl.ds(sid * NPT, NPT)],
                        agg_out.at[cid, sid])

    out_type = (jax.ShapeDtypeStruct((NC, NS, NPT, H), _f32),)
    if write_h:
        out_type = out_type + (jax.ShapeDtypeStruct((E, H), _f32),)
    return pl.kernel(
        body,
        out_type=out_type,
        mesh=plsc.VectorSubcoreMesh(**_SC_MESH),
        compiler_params=pltpu.CompilerParams(needs_layout_passes=False),
        scratch_types=[
            pltpu.VMEM((2, KB), jnp.int32),
            pltpu.VMEM((2, KB), jnp.int32),
            pltpu.VMEM((2, KB), jnp.int32),
            pltpu.VMEM((2, KB), _f32),
            pltpu.VMEM((2, KB, H), _f32),
            pltpu.VMEM((2, KB, H), _f32),
            pltpu.VMEM_SHARED((N, H), _f32),
            pltpu.SemaphoreType.DMA((2,)),
            pltpu.SemaphoreType.DMA((2,)),
            pltpu.SemaphoreType.DMA((2,)),
            pltpu.SemaphoreType.DMA((2,)),
            pltpu.SemaphoreType.DMA((2,)),
        ],
    )


def _dot(a, b, precision=None):
    return jnp.dot(a, b, preferred_element_type=_f32, precision=precision)


def _tc_edge_prologue_body(ef_ref, aw_ref, s_ref, we_ref, be_ref, wib_ref,
                           e2_ref, scale_ref):
    e = jnp.maximum(_dot(ef_ref[...], we_ref[...]) + be_ref[...], 0.0)
    e2_ref[...] = _dot(e, wib_ref[...])
    aw = aw_ref[...]
    diss = jnp.where(aw == 0.0, _f32(-1.0),
                     jnp.log(jnp.maximum(aw, 1e-12)) * 2.0)
    ldec = jnp.where(aw == 1.0, _f32(1.0), diss)
    dist = jnp.sqrt(s_ref[...] + 1e-12)
    scale_ref[...] = ldec * jnp.exp(-dist)


def _tc_edge_prologue(ef, aw3, s3, we, be, wib):
    return pl.pallas_call(
        _tc_edge_prologue_body,
        grid=(GE,),
        in_specs=[
            pl.BlockSpec((BE, 14), lambda i: (i, 0)),
            pl.BlockSpec((1, 1, BE), lambda i: (i, 0, 0)),
            pl.BlockSpec((1, 1, BE), lambda i: (i, 0, 0)),
            pl.BlockSpec((14, HE), lambda i: (0, 0)),
            pl.BlockSpec((1, HE), lambda i: (0, 0)),
            pl.BlockSpec((HE, H), lambda i: (0, 0)),
        ],
        out_specs=[
            pl.BlockSpec((BE, H), lambda i: (i, 0)),
            pl.BlockSpec((1, 1, BE), lambda i: (i, 0, 0)),
        ],
        out_shape=[
            jax.ShapeDtypeStruct((E, H), _f32),
            jax.ShapeDtypeStruct((GE, 1, BE), _f32),
        ],
        compiler_params=pltpu.CompilerParams(
            dimension_semantics=("parallel",)),
    )(ef, aw3, s3, we, be, wib)


def _tc_node_prologue_body(a_ref, degp_ref, wa_ref, ba_ref, emb_ref, wit_ref,
                           x_ref, c_ref):
    deg = jnp.sum(degp_ref[...], axis=(0, 2)) * _f32(1.0 / H)
    degi = jnp.clip(deg.astype(jnp.int32), 0, 199)
    oh = (degi[:, None] ==
          lax.broadcasted_iota(jnp.int32, (BN, 200), 1)).astype(_f32)
    xb = jnp.maximum(_dot(a_ref[...], wa_ref[...]) + ba_ref[...], 0.0)
    xb = xb + _dot(oh, emb_ref[...], precision=lax.Precision.HIGHEST)
    x_ref[...] = xb
    c_ref[...] = _dot(xb, wit_ref[...])


def _tc_node_prologue(a, degp, wa, ba, emb, wit):
    return pl.pallas_call(
        _tc_node_prologue_body,
        grid=(GN,),
        in_specs=[
            pl.BlockSpec((BN, 70), lambda i: (i, 0)),
            pl.BlockSpec((NC, BN, H), lambda i: (0, i, 0)),
            pl.BlockSpec((70, H), lambda i: (0, 0)),
            pl.BlockSpec((1, H), lambda i: (0, 0)),
            pl.BlockSpec((200, H), lambda i: (0, 0)),
            pl.BlockSpec((H, H), lambda i: (0, 0)),
        ],
        out_specs=[
            pl.BlockSpec((BN, H), lambda i: (i, 0)),
            pl.BlockSpec((BN, H), lambda i: (i, 0)),
        ],
        out_shape=[
            jax.ShapeDtypeStruct((N, H), _f32),
            jax.ShapeDtypeStruct((N, H), _f32),
        ],
        compiler_params=pltpu.CompilerParams(
            dimension_semantics=("parallel",)),
    )(a, degp, wa, ba, emb, wit)


def _tc_mid_body(aggp_ref, wh_ref, b_ref):
    b_ref[...] = _dot(aggp_ref[0] + aggp_ref[1], wh_ref[...])


def _tc_mid(aggp, wh):
    return pl.pallas_call(
        _tc_mid_body,
        grid=(GN,),
        in_specs=[
            pl.BlockSpec((NC, BN, H), lambda i: (0, i, 0)),
            pl.BlockSpec((H, H), lambda i: (0, 0)),
        ],
        out_specs=pl.BlockSpec((BN, H), lambda i: (i, 0)),
        out_shape=jax.ShapeDtypeStruct((N, H), _f32),
        compiler_params=pltpu.CompilerParams(
            dimension_semantics=("parallel",)),
    )(aggp, wh)


def _tc_epilogue_body(x_ref, aggp_ref, wot_ref, wob_ref, w1_ref, b1_ref,
                      w2_ref, b2_ref, w3_ref, b3_ref, w4_ref, b4_ref,
                      out_ref, acc_ref):
    i = pl.program_id(0)

    @pl.when(i == 0)
    def _():
        acc_ref[...] = jnp.zeros_like(acc_ref)

    hn = aggp_ref[0] + aggp_ref[1]
    hv = jnp.maximum(_dot(x_ref[...], wot_ref[...]) +
                     _dot(hn, wob_ref[...]), 0.0)
    acc_ref[...] += jnp.sum(hv, axis=0, keepdims=True)

    @pl.when(i == GN - 1)
    def _():
        g = acc_ref[...] * _f32(1.0 / N)
        z = jnp.maximum(_dot(g, w1_ref[...]) + b1_ref[...], 0.0)
        z = jnp.maximum(_dot(z, w2_ref[...]) + b2_ref[...], 0.0)
        z = jnp.maximum(_dot(z, w3_ref[...]) + b3_ref[...], 0.0)
        out_ref[...] = _dot(z, w4_ref[...]) + b4_ref[...]


def _tc_epilogue(x, aggp, wot, wob, w1, b1, w2, b2, w3, b3, w4, b4):
    full = lambda *shape: pl.BlockSpec(shape, lambda i: (0,) * len(shape))
    return pl.pallas_call(
        _tc_epilogue_body,
        grid=(GN,),
        in_specs=[
            pl.BlockSpec((BN, H), lambda i: (i, 0)),
            pl.BlockSpec((NC, BN, H), lambda i: (0, i, 0)),
            full(H, H), full(H, H),
            full(H, H // 2), full(1, H // 2),
            full(H // 2, H // 4), full(1, H // 4),
            full(H // 4, H // 8), full(1, H // 8),
            full(H // 8, 1), full(1, 1),
        ],
        out_specs=pl.BlockSpec((1, 1), lambda i: (0, 0)),
        out_shape=jax.ShapeDtypeStruct((1, 1), _f32),
        scratch_shapes=[pltpu.VMEM((1, H), _f32)],
        compiler_params=pltpu.CompilerParams(
            dimension_semantics=("arbitrary",)),
    )(x, aggp, wot, wob, w1, b1, w2, b2, w3, b3, w4, b4)


def kernel(atom_feature, atom_coordinate, edge_feature, attention_weight,
           edge_index, W_atom, b_atom, W_edge, b_edge, degree_emb, W_i, W_h,
           W_o, W1, b1, W2, b2, W3, b3, W4, b4):
    src2d = edge_index[0].reshape(NW, NB, KB)
    dst2d = edge_index[1].reshape(NW, NB, KB)
    cx = atom_coordinate[:, 0]
    cy = atom_coordinate[:, 1]
    cz = atom_coordinate[:, 2]
    onesH = jnp.ones((KB, H), _f32)
    zerosH = jnp.zeros((NPT, H), _f32)

    degp, s = _sc_pass0()(cx, cy, cz, src2d, dst2d, zerosH, onesH)
    degp = degp.reshape(NC, N, H)

    e2, scale3 = _tc_edge_prologue(
        edge_feature,
        attention_weight.reshape(GE, 1, BE),
        s.reshape(GE, 1, BE),
        W_edge, b_edge.reshape(1, HE), W_i[H:])
    scale = scale3.reshape(NW, NB, KB)

    x, c = _tc_node_prologue(
        atom_feature, degp, W_atom, b_atom.reshape(1, H), degree_emb,
        W_i[:H])

    aggp, h0 = _make_edge_pass(True)(e2, c, src2d, dst2d, scale, zerosH)
    for _ in range(RADIUS):
        b = _tc_mid(aggp.reshape(NC, N, H), W_h)
        aggp, = _make_edge_pass(False)(h0, b, src2d, dst2d, scale, zerosH)

    out = _tc_epilogue(
        x, aggp.reshape(NC, N, H), W_o[:H], W_o[H:], W1, b1.reshape(1, H // 2),
        W2, b2.reshape(1, H // 4), W3, b3.reshape(1, H // 8),
        W4, b4.reshape(1, 1))
    return out.reshape(1)


# speedup vs baseline: 1.0337x; 1.0337x over previous
import functools

import jax
import jax.numpy as jnp
from jax import lax
from jax.experimental import pallas as pl
from jax.experimental.pallas import tpu as pltpu
from jax.experimental.pallas import tpu_sc as plsc

N = 10000
E = 320000
H = 128
HE = 64
RADIUS = 3

NC = 2
NS = 16
NW = NC * NS
EC = E // NW
KB = 80
NB = EC // KB
NPT = N // NS
VPR = H // 16

_SC_MESH = dict(core_axis_name="c", subcore_axis_name="s",
                num_cores=NC, num_subcores=NS)

BE = 512
GE = E // BE
BN = 2000
GN = N // BN

_f32 = jnp.float32


def _sc_pass0_body(cx_hbm, cy_hbm, cz_hbm, src_hbm, dst_hbm, zerosN_hbm,
                   deg_out, s_out,
                   cx_v, cy_v, cz_v, src_v, dst_v, s_v, deg_t):
    cid = lax.axis_index("c")
    sid = lax.axis_index("s")
    wid = sid * NC + cid
    pltpu.sync_copy(cx_hbm, cx_v)
    pltpu.sync_copy(cy_hbm, cy_v)
    pltpu.sync_copy(cz_hbm, cz_v)
    pltpu.sync_copy(zerosN_hbm, deg_t)
    ones16 = jnp.ones((16,), _f32)

    @pl.loop(0, NB)
    def _blk(b):
        pltpu.sync_copy(src_hbm.at[wid, b], src_v)
        pltpu.sync_copy(dst_hbm.at[wid, b], dst_v)
        for j in range(KB // 16):
            si = src_v[pl.ds(j * 16, 16)]
            di = dst_v[pl.ds(j * 16, 16)]
            acc = jnp.zeros((16,), _f32)
            for cv in (cx_v, cy_v, cz_v):
                d = plsc.load_gather(cv, [si]) - plsc.load_gather(cv, [di])
                acc = acc + d * d
            s_v[pl.ds(j * 16, 16)] = acc
            plsc.addupdate_scatter(deg_t, [di], ones16)
        pltpu.sync_copy(s_v, s_out.at[wid, b])

    pltpu.sync_copy(deg_t, deg_out.at[cid, sid])


@functools.cache
def _sc_pass0():
    return pl.kernel(
        _sc_pass0_body,
        out_type=(jax.ShapeDtypeStruct((NC, NS, N), _f32),
                  jax.ShapeDtypeStruct((NW, NB, KB), _f32)),
        mesh=plsc.VectorSubcoreMesh(**_SC_MESH),
        compiler_params=pltpu.CompilerParams(needs_layout_passes=False),
        scratch_types=[
            pltpu.VMEM((N,), _f32),
            pltpu.VMEM((N,), _f32),
            pltpu.VMEM((N,), _f32),
            pltpu.VMEM((KB,), jnp.int32),
            pltpu.VMEM((KB,), jnp.int32),
            pltpu.VMEM((KB,), _f32),
            pltpu.VMEM((N,), _f32),
        ],
    )


@functools.cache
def _make_edge_pass(write_h, mode="add"):
    def body(lin_hbm, tab_hbm, src_hbm, dst_hbm, scale_hbm, zeros_hbm, *rest):
        if write_h:
            (agg_out, h_out, src_v, dst_v, dsts_v, scale_v, lin_v, gath_v,
             agg_sh, sem_idx, sem_lin, sem_g, sem_sc, sem_h) = rest
        else:
            (agg_out, src_v, dst_v, dsts_v, scale_v, lin_v, gath_v,
             agg_sh, sem_idx, sem_lin, sem_g, sem_sc, sem_h) = rest
        cid = lax.axis_index("c")
        sid = lax.axis_index("s")
        wid = sid * NC + cid
        pltpu.sync_copy(zeros_hbm, agg_sh.at[pl.ds(sid * NPT, NPT)])
        plsc.subcore_barrier()
        ebase = wid * EC

        def lin_slice(b):
            return lin_hbm.at[pl.ds(ebase + b * KB, KB)]

        def issue_idx(b, k):
            pltpu.async_copy(src_hbm.at[wid, b], src_v.at[k], sem_idx.at[k])
            pltpu.async_copy(dst_hbm.at[wid, b], dst_v.at[k], sem_idx.at[k])
            pltpu.async_copy(scale_hbm.at[wid, b], scale_v.at[k],
                             sem_idx.at[k])

        def wait_idx(b, k):
            pltpu.make_async_copy(src_hbm.at[wid, b], src_v.at[k],
                                  sem_idx.at[k]).wait()
            pltpu.make_async_copy(dst_hbm.at[wid, b], dst_v.at[k],
                                  sem_idx.at[k]).wait()
            pltpu.make_async_copy(scale_hbm.at[wid, b], scale_v.at[k],
                                  sem_idx.at[k]).wait()

        def issue_lin(b, k):
            pltpu.async_copy(lin_slice(b), lin_v.at[k], sem_lin.at[k])

        def issue_gather(k):
            pltpu.async_copy(tab_hbm.at[src_v.at[k]], gath_v.at[k],
                             sem_g.at[k])

        def wait_gather(k):
            pltpu.make_async_copy(tab_hbm.at[src_v.at[k]], gath_v.at[k],
                                  sem_g.at[k]).wait()

        def issue_scatter(k):
            if mode == "add":
                pltpu.async_copy(gath_v.at[k], agg_sh.at[dsts_v.at[k]],
                                 sem_sc.at[k], add=True)
            elif mode == "linear":
                pltpu.async_copy(gath_v.at[k], agg_sh.at[pl.ds(k * KB, KB)],
                                 sem_sc.at[k])

        def wait_scatter(k):
            if mode == "add":
                pltpu.make_async_copy(gath_v.at[k], agg_sh.at[dsts_v.at[k]],
                                      sem_sc.at[k]).wait()
            elif mode == "linear":
                pltpu.make_async_copy(gath_v.at[k],
                                      agg_sh.at[pl.ds(k * KB, KB)],
                                      sem_sc.at[k]).wait()

        def block(b, k, tail):
            wait_gather(k)
            for j in range(KB // 16):
                sl = pl.ds(j * 16, 16)
                dsts_v.at[k][sl] = dst_v.at[k][sl]
            pltpu.make_async_copy(lin_slice(b), lin_v.at[k],
                                  sem_lin.at[k]).wait()
            lin_k = lin_v.at[k]
            gath_k = gath_v.at[k]
            scale_k = scale_v.at[k]

            @pl.loop(0, KB, unroll=2)
            def _edge(e):
                sc = plsc.load_gather(scale_k,
                                      [jnp.full((16,), e, jnp.int32)])
                sls = [pl.ds(v * 16, 16) for v in range(VPR)]
                lins = [lin_k[e, sl] for sl in sls]
                gats = [gath_k[e, sl] for sl in sls]
                hs = [jnp.maximum(a + b, 0.0) for a, b in zip(lins, gats)]
                vals = [h * sc for h in hs]
                for v in range(VPR):
                    if write_h:
                        lin_k[e, sls[v]] = hs[v]
                    gath_k[e, sls[v]] = vals[v]

            if not tail:
                @pl.when(b + 2 < NB)
                def _():
                    issue_idx(b + 2, k)
                    if not write_h:
                        issue_lin(b + 2, k)
            if not tail:
                @pl.when(b >= 1)
                def _():
                    wait_scatter(k ^ 1)
                wait_idx(b + 1, k ^ 1)
                issue_gather(k ^ 1)
            issue_scatter(k)
            if write_h:
                pltpu.async_copy(lin_k, h_out.at[pl.ds(ebase + b * KB, KB)],
                                 sem_h.at[k])
                pltpu.make_async_copy(lin_k,
                                      h_out.at[pl.ds(ebase + b * KB, KB)],
                                      sem_h.at[k]).wait()
                if not tail:
                    @pl.when(b + 2 < NB)
                    def _():
                        issue_lin(b + 2, k)

        issue_idx(0, 0)
        issue_idx(1, 1)
        issue_lin(0, 0)
        issue_lin(1, 1)
        wait_idx(0, 0)
        issue_gather(0)

        @pl.loop(0, NB - 1, step=2)
        def _pair(b0):
            block(b0, 0, False)
            block(b0 + 1, 1, False)

        block(NB - 1, 0, True)
        wait_scatter(0)
        wait_scatter(1)

        plsc.subcore_barrier()
        pltpu.sync_copy(agg_sh.at[pl.ds(sid * NPT, NPT)],
                        agg_out.at[cid, sid])

    out_type = (jax.ShapeDtypeStruct((NC, NS, NPT, H), _f32),)
    if write_h:
        out_type = out_type + (jax.ShapeDtypeStruct((E, H), _f32),)
    return pl.kernel(
        body,
        out_type=out_type,
        mesh=plsc.VectorSubcoreMesh(**_SC_MESH),
        compiler_params=pltpu.CompilerParams(needs_layout_passes=False),
        scratch_types=[
            pltpu.VMEM((2, KB), jnp.int32),
            pltpu.VMEM((2, KB), jnp.int32),
            pltpu.VMEM((2, KB), jnp.int32),
            pltpu.VMEM((2, KB), _f32),
            pltpu.VMEM((2, KB, H), _f32),
            pltpu.VMEM((2, KB, H), _f32),
            pltpu.VMEM_SHARED((N, H), _f32),
            pltpu.SemaphoreType.DMA((2,)),
            pltpu.SemaphoreType.DMA((2,)),
            pltpu.SemaphoreType.DMA((2,)),
            pltpu.SemaphoreType.DMA((2,)),
            pltpu.SemaphoreType.DMA((2,)),
        ],
    )


def _dot(a, b, precision=None):
    return jnp.dot(a, b, preferred_element_type=_f32, precision=precision)


def _tc_edge_prologue_body(ef_ref, aw_ref, s_ref, we_ref, be_ref, wib_ref,
                           e2_ref, scale_ref):
    e = jnp.maximum(_dot(ef_ref[...], we_ref[...]) + be_ref[...], 0.0)
    e2_ref[...] = _dot(e, wib_ref[...])
    aw = aw_ref[...]
    diss = jnp.where(aw == 0.0, _f32(-1.0),
                     jnp.log(jnp.maximum(aw, 1e-12)) * 2.0)
    ldec = jnp.where(aw == 1.0, _f32(1.0), diss)
    dist = jnp.sqrt(s_ref[...] + 1e-12)
    scale_ref[...] = ldec * jnp.exp(-dist)


def _tc_edge_prologue(ef, aw3, s3, we, be, wib):
    return pl.pallas_call(
        _tc_edge_prologue_body,
        grid=(GE,),
        in_specs=[
            pl.BlockSpec((BE, 14), lambda i: (i, 0)),
            pl.BlockSpec((1, 1, BE), lambda i: (i, 0, 0)),
            pl.BlockSpec((1, 1, BE), lambda i: (i, 0, 0)),
            pl.BlockSpec((14, HE), lambda i: (0, 0)),
            pl.BlockSpec((1, HE), lambda i: (0, 0)),
            pl.BlockSpec((HE, H), lambda i: (0, 0)),
        ],
        out_specs=[
            pl.BlockSpec((BE, H), lambda i: (i, 0)),
            pl.BlockSpec((1, 1, BE), lambda i: (i, 0, 0)),
        ],
        out_shape=[
            jax.ShapeDtypeStruct((E, H), _f32),
            jax.ShapeDtypeStruct((GE, 1, BE), _f32),
        ],
        compiler_params=pltpu.CompilerParams(
            dimension_semantics=("parallel",)),
    )(ef, aw3, s3, we, be, wib)


def _tc_node_prologue_body(a_ref, degp_ref, wa_ref, ba_ref, emb_ref, wit_ref,
                           x_ref, c_ref):
    deg = jnp.sum(degp_ref[...], axis=1)
    degi = jnp.clip(deg.astype(jnp.int32), 0, 199)
    oh = (degi[:, None] ==
          lax.broadcasted_iota(jnp.int32, (BN, 200), 1)).astype(_f32)
    xb = jnp.maximum(_dot(a_ref[...], wa_ref[...]) + ba_ref[...], 0.0)
    xb = xb + _dot(oh, emb_ref[...], precision=lax.Precision.HIGHEST)
    x_ref[...] = xb
    c_ref[...] = _dot(xb, wit_ref[...])


def _tc_node_prologue(a, degp, wa, ba, emb, wit):
    return pl.pallas_call(
        _tc_node_prologue_body,
        grid=(GN,),
        in_specs=[
            pl.BlockSpec((BN, 70), lambda i: (i, 0)),
            pl.BlockSpec((BN, NW), lambda i: (i, 0)),
            pl.BlockSpec((70, H), lambda i: (0, 0)),
            pl.BlockSpec((1, H), lambda i: (0, 0)),
            pl.BlockSpec((200, H), lambda i: (0, 0)),
            pl.BlockSpec((H, H), lambda i: (0, 0)),
        ],
        out_specs=[
            pl.BlockSpec((BN, H), lambda i: (i, 0)),
            pl.BlockSpec((BN, H), lambda i: (i, 0)),
        ],
        out_shape=[
            jax.ShapeDtypeStruct((N, H), _f32),
            jax.ShapeDtypeStruct((N, H), _f32),
        ],
        compiler_params=pltpu.CompilerParams(
            dimension_semantics=("parallel",)),
    )(a, degp, wa, ba, emb, wit)


def _tc_mid_body(aggp_ref, wh_ref, b_ref):
    b_ref[...] = _dot(aggp_ref[0] + aggp_ref[1], wh_ref[...])


def _tc_mid(aggp, wh):
    return pl.pallas_call(
        _tc_mid_body,
        grid=(GN,),
        in_specs=[
            pl.BlockSpec((NC, BN, H), lambda i: (0, i, 0)),
            pl.BlockSpec((H, H), lambda i: (0, 0)),
        ],
        out_specs=pl.BlockSpec((BN, H), lambda i: (i, 0)),
        out_shape=jax.ShapeDtypeStruct((N, H), _f32),
        compiler_params=pltpu.CompilerParams(
            dimension_semantics=("parallel",)),
    )(aggp, wh)


def _tc_epilogue_body(x_ref, aggp_ref, wot_ref, wob_ref, w1_ref, b1_ref,
                      w2_ref, b2_ref, w3_ref, b3_ref, w4_ref, b4_ref,
                      out_ref, acc_ref):
    i = pl.program_id(0)

    @pl.when(i == 0)
    def _():
        acc_ref[...] = jnp.zeros_like(acc_ref)

    hn = aggp_ref[0] + aggp_ref[1]
    hv = jnp.maximum(_dot(x_ref[...], wot_ref[...]) +
                     _dot(hn, wob_ref[...]), 0.0)
    acc_ref[...] += jnp.sum(hv, axis=0, keepdims=True)

    @pl.when(i == GN - 1)
    def _():
        g = acc_ref[...] * _f32(1.0 / N)
        z = jnp.maximum(_dot(g, w1_ref[...]) + b1_ref[...], 0.0)
        z = jnp.maximum(_dot(z, w2_ref[...]) + b2_ref[...], 0.0)
        z = jnp.maximum(_dot(z, w3_ref[...]) + b3_ref[...], 0.0)
        out_ref[...] = _dot(z, w4_ref[...]) + b4_ref[...]


def _tc_epilogue(x, aggp, wot, wob, w1, b1, w2, b2, w3, b3, w4, b4):
    full = lambda *shape: pl.BlockSpec(shape, lambda i: (0,) * len(shape))
    return pl.pallas_call(
        _tc_epilogue_body,
        grid=(GN,),
        in_specs=[
            pl.BlockSpec((BN, H), lambda i: (i, 0)),
            pl.BlockSpec((NC, BN, H), lambda i: (0, i, 0)),
            full(H, H), full(H, H),
            full(H, H // 2), full(1, H // 2),
            full(H // 2, H // 4), full(1, H // 4),
            full(H // 4, H // 8), full(1, H // 8),
            full(H // 8, 1), full(1, 1),
        ],
        out_specs=pl.BlockSpec((1, 1), lambda i: (0, 0)),
        out_shape=jax.ShapeDtypeStruct((1, 1), _f32),
        scratch_shapes=[pltpu.VMEM((1, H), _f32)],
        compiler_params=pltpu.CompilerParams(
            dimension_semantics=("arbitrary",)),
    )(x, aggp, wot, wob, w1, b1, w2, b2, w3, b3, w4, b4)


def kernel(atom_feature, atom_coordinate, edge_feature, attention_weight,
           edge_index, W_atom, b_atom, W_edge, b_edge, degree_emb, W_i, W_h,
           W_o, W1, b1, W2, b2, W3, b3, W4, b4):
    src2d = edge_index[0].reshape(NW, NB, KB)
    dst2d = edge_index[1].reshape(NW, NB, KB)
    cx = atom_coordinate[:, 0]
    cy = atom_coordinate[:, 1]
    cz = atom_coordinate[:, 2]
    zerosH = jnp.zeros((NPT, H), _f32)
    zerosN = jnp.zeros((N,), _f32)

    degp, s = _sc_pass0()(cx, cy, cz, src2d, dst2d, zerosN)
    degp = degp.reshape(NC * NS, N).T

    e2, scale3 = _tc_edge_prologue(
        edge_feature,
        attention_weight.reshape(GE, 1, BE),
        s.reshape(GE, 1, BE),
        W_edge, b_edge.reshape(1, HE), W_i[H:])
    scale = scale3.reshape(NW, NB, KB)

    x, c = _tc_node_prologue(
        atom_feature, degp, W_atom, b_atom.reshape(1, H), degree_emb,
        W_i[:H])

    aggp, h0 = _make_edge_pass(True)(e2, c, src2d, dst2d, scale, zerosH)
    for _ in range(RADIUS):
        b = _tc_mid(aggp.reshape(NC, N, H), W_h)
        aggp, = _make_edge_pass(False)(h0, b, src2d, dst2d, scale, zerosH)

    out = _tc_epilogue(
        x, aggp.reshape(NC, N, H), W_o[:H], W_o[H:], W1, b1.reshape(1, H // 2),
        W2, b2.reshape(1, H // 4), W3, b3.reshape(1, H // 8),
        W4, b4.reshape(1, 1))
    return out.reshape(1)


# speedup vs baseline: 1.0773x; 1.0422x over previous
import functools

import jax
import jax.numpy as jnp
from jax import lax
from jax.experimental import pallas as pl
from jax.experimental.pallas import tpu as pltpu
from jax.experimental.pallas import tpu_sc as plsc

N = 10000
E = 320000
H = 128
HE = 64
RADIUS = 3

NC = 2
NS = 16
NW = NC * NS
EC = E // NW
KB = 80
NB = EC // KB
NPT = N // NS
VPR = H // 16

_SC_MESH = dict(core_axis_name="c", subcore_axis_name="s",
                num_cores=NC, num_subcores=NS)

BE = 512
GE = E // BE
BN = 2000
GN = N // BN

_f32 = jnp.float32


def _sc_pass0_body(cx_hbm, cy_hbm, cz_hbm, src_hbm, dst_hbm, zerosN_hbm,
                   deg_out, s_out,
                   cx_v, cy_v, cz_v, src_v, dst_v, s_v, deg_t):
    cid = lax.axis_index("c")
    sid = lax.axis_index("s")
    wid = sid * NC + cid
    pltpu.sync_copy(cx_hbm, cx_v)
    pltpu.sync_copy(cy_hbm, cy_v)
    pltpu.sync_copy(cz_hbm, cz_v)
    pltpu.sync_copy(zerosN_hbm, deg_t)
    ones16 = jnp.ones((16,), _f32)

    @pl.loop(0, NB)
    def _blk(b):
        pltpu.sync_copy(src_hbm.at[wid, b], src_v)
        pltpu.sync_copy(dst_hbm.at[wid, b], dst_v)
        for j in range(KB // 16):
            si = src_v[pl.ds(j * 16, 16)]
            di = dst_v[pl.ds(j * 16, 16)]
            acc = jnp.zeros((16,), _f32)
            for cv in (cx_v, cy_v, cz_v):
                d = plsc.load_gather(cv, [si]) - plsc.load_gather(cv, [di])
                acc = acc + d * d
            s_v[pl.ds(j * 16, 16)] = acc
            plsc.addupdate_scatter(deg_t, [di], ones16)
        pltpu.sync_copy(s_v, s_out.at[wid, b])

    pltpu.sync_copy(deg_t, deg_out.at[cid, sid])


@functools.cache
def _sc_pass0():
    return pl.kernel(
        _sc_pass0_body,
        out_type=(jax.ShapeDtypeStruct((NC, NS, N), _f32),
                  jax.ShapeDtypeStruct((NW, NB, KB), _f32)),
        mesh=plsc.VectorSubcoreMesh(**_SC_MESH),
        compiler_params=pltpu.CompilerParams(needs_layout_passes=False),
        scratch_types=[
            pltpu.VMEM((N,), _f32),
            pltpu.VMEM((N,), _f32),
            pltpu.VMEM((N,), _f32),
            pltpu.VMEM((KB,), jnp.int32),
            pltpu.VMEM((KB,), jnp.int32),
            pltpu.VMEM((KB,), _f32),
            pltpu.VMEM((N,), _f32),
        ],
    )


@functools.cache
def _make_edge_pass(write_h, mode="add"):
    def body(lin_hbm, tab_hbm, src_hbm, dst_hbm, scale_hbm, zeros_hbm, *rest):
        if write_h:
            (agg_out, h_out, src_v, dst_v, dsts_v, scale_v, lin_v, gath_v,
             agg_sh, sem_idx, sem_lin, sem_g, sem_sc, sem_h) = rest
        else:
            (agg_out, src_v, dst_v, dsts_v, scale_v, lin_v, gath_v,
             agg_sh, sem_idx, sem_lin, sem_g, sem_sc, sem_h) = rest
        cid = lax.axis_index("c")
        sid = lax.axis_index("s")
        wid = sid * NC + cid
        pltpu.sync_copy(zeros_hbm, agg_sh.at[pl.ds(sid * NPT, NPT)])
        plsc.subcore_barrier()
        ebase = wid * EC

        def lin_slice(b):
            return lin_hbm.at[pl.ds(ebase + b * KB, KB)]

        def issue_idx(b, k):
            pltpu.async_copy(src_hbm.at[wid, b], src_v.at[k], sem_idx.at[k])
            pltpu.async_copy(dst_hbm.at[wid, b], dst_v.at[k], sem_idx.at[k])
            pltpu.async_copy(scale_hbm.at[wid, b], scale_v.at[k],
                             sem_idx.at[k])

        def wait_idx(b, k):
            pltpu.make_async_copy(src_hbm.at[wid, b], src_v.at[k],
                                  sem_idx.at[k]).wait()
            pltpu.make_async_copy(dst_hbm.at[wid, b], dst_v.at[k],
                                  sem_idx.at[k]).wait()
            pltpu.make_async_copy(scale_hbm.at[wid, b], scale_v.at[k],
                                  sem_idx.at[k]).wait()

        def issue_lin(b, k):
            pltpu.async_copy(lin_slice(b), lin_v.at[k], sem_lin.at[k])

        def issue_gather(k):
            pltpu.async_copy(tab_hbm.at[src_v.at[k]], gath_v.at[k],
                             sem_g.at[k])

        def wait_gather(k):
            pltpu.make_async_copy(tab_hbm.at[src_v.at[k]], gath_v.at[k],
                                  sem_g.at[k]).wait()

        def issue_scatter(k):
            if mode == "add":
                pltpu.async_copy(gath_v.at[k], agg_sh.at[dsts_v.at[k]],
                                 sem_sc.at[k], add=True)
            elif mode == "linear":
                pltpu.async_copy(gath_v.at[k], agg_sh.at[pl.ds(k * KB, KB)],
                                 sem_sc.at[k])

        def wait_scatter(k):
            if mode == "add":
                pltpu.make_async_copy(gath_v.at[k], agg_sh.at[dsts_v.at[k]],
                                      sem_sc.at[k]).wait()
            elif mode == "linear":
                pltpu.make_async_copy(gath_v.at[k],
                                      agg_sh.at[pl.ds(k * KB, KB)],
                                      sem_sc.at[k]).wait()

        def block(b, k, tail):
            wait_gather(k)
            for j in range(KB // 16):
                sl = pl.ds(j * 16, 16)
                dsts_v.at[k][sl] = dst_v.at[k][sl]
            pltpu.make_async_copy(lin_slice(b), lin_v.at[k],
                                  sem_lin.at[k]).wait()
            lin_k = lin_v.at[k]
            gath_k = gath_v.at[k]
            scale_k = scale_v.at[k]

            @pl.loop(0, KB, unroll=2)
            def _edge(e):
                sc = plsc.load_gather(scale_k,
                                      [jnp.full((16,), e, jnp.int32)])
                sls = [pl.ds(v * 16, 16) for v in range(VPR)]
                lins = [lin_k[e, sl] for sl in sls]
                gats = [gath_k[e, sl] for sl in sls]
                hs = [jnp.maximum(a + b, 0.0) for a, b in zip(lins, gats)]
                vals = [h * sc for h in hs]
                for v in range(VPR):
                    if write_h:
                        lin_k[e, sls[v]] = hs[v]
                    gath_k[e, sls[v]] = vals[v]

            if not tail:
                @pl.when(b + 2 < NB)
                def _():
                    issue_idx(b + 2, k)
                    if not write_h:
                        issue_lin(b + 2, k)
            if not tail:
                @pl.when(b >= 1)
                def _():
                    wait_scatter(k ^ 1)
                wait_idx(b + 1, k ^ 1)
                issue_gather(k ^ 1)
            issue_scatter(k)
            if write_h:
                pltpu.async_copy(lin_k, h_out.at[pl.ds(ebase + b * KB, KB)],
                                 sem_h.at[k])
                pltpu.make_async_copy(lin_k,
                                      h_out.at[pl.ds(ebase + b * KB, KB)],
                                      sem_h.at[k]).wait()
                if not tail:
                    @pl.when(b + 2 < NB)
                    def _():
                        issue_lin(b + 2, k)

        issue_idx(0, 0)
        issue_idx(1, 1)
        issue_lin(0, 0)
        issue_lin(1, 1)
        wait_idx(0, 0)
        issue_gather(0)

        @pl.loop(0, NB - 1, step=2)
        def _pair(b0):
            block(b0, 0, False)
            block(b0 + 1, 1, False)

        block(NB - 1, 0, True)
        wait_scatter(0)
        wait_scatter(1)

        plsc.subcore_barrier()
        pltpu.sync_copy(agg_sh.at[pl.ds(sid * NPT, NPT)],
                        agg_out.at[cid, sid])

    out_type = (jax.ShapeDtypeStruct((NC, NS, NPT, H), _f32),)
    if write_h:
        out_type = out_type + (jax.ShapeDtypeStruct((E, H), _f32),)
    return pl.kernel(
        body,
        out_type=out_type,
        mesh=plsc.VectorSubcoreMesh(**_SC_MESH),
        compiler_params=pltpu.CompilerParams(needs_layout_passes=False),
        scratch_types=[
            pltpu.VMEM((2, KB), jnp.int32),
            pltpu.VMEM((2, KB), jnp.int32),
            pltpu.VMEM((2, KB), jnp.int32),
            pltpu.VMEM((2, KB), _f32),
            pltpu.VMEM((2, KB, H), _f32),
            pltpu.VMEM((2, KB, H), _f32),
            pltpu.VMEM_SHARED((N, H), _f32),
            pltpu.SemaphoreType.DMA((2,)),
            pltpu.SemaphoreType.DMA((2,)),
            pltpu.SemaphoreType.DMA((2,)),
            pltpu.SemaphoreType.DMA((2,)),
            pltpu.SemaphoreType.DMA((2,)),
        ],
    )


def _dot(a, b, precision=None):
    return jnp.dot(a, b, preferred_element_type=_f32, precision=precision)


def _tc_edge_mm_body(ef_ref, we_ref, be_ref, wib_ref, e2_ref):
    e = jnp.maximum(_dot(ef_ref[...], we_ref[...]) + be_ref[...], 0.0)
    e2_ref[...] = _dot(e, wib_ref[...])


def _tc_edge_mm(ef, we, be, wib):
    return pl.pallas_call(
        _tc_edge_mm_body,
        grid=(GE,),
        in_specs=[
            pl.BlockSpec((BE, 14), lambda i: (i, 0)),
            pl.BlockSpec((14, HE), lambda i: (0, 0)),
            pl.BlockSpec((1, HE), lambda i: (0, 0)),
            pl.BlockSpec((HE, H), lambda i: (0, 0)),
        ],
        out_specs=pl.BlockSpec((BE, H), lambda i: (i, 0)),
        out_shape=jax.ShapeDtypeStruct((E, H), _f32),
        compiler_params=pltpu.CompilerParams(
            dimension_semantics=("parallel",)),
    )(ef, we, be, wib)


def _tc_edge_scale_body(aw_ref, s_ref, scale_ref):
    aw = aw_ref[...]
    diss = jnp.where(aw == 0.0, _f32(-1.0),
                     jnp.log(jnp.maximum(aw, 1e-12)) * 2.0)
    ldec = jnp.where(aw == 1.0, _f32(1.0), diss)
    dist = jnp.sqrt(s_ref[...] + 1e-12)
    scale_ref[...] = ldec * jnp.exp(-dist)


def _tc_edge_scale(aw2, s2):
    full2 = pl.BlockSpec((E // H, H), lambda: (0, 0))
    return pl.pallas_call(
        _tc_edge_scale_body,
        in_specs=[full2, full2],
        out_specs=full2,
        out_shape=jax.ShapeDtypeStruct((E // H, H), _f32),
    )(aw2, s2)


def _tc_node_prologue_body(a_ref, degp_ref, wa_ref, ba_ref, emb_ref, wit_ref,
                           x_ref, c_ref):
    deg = jnp.sum(degp_ref[...], axis=1)
    degi = jnp.clip(deg.astype(jnp.int32), 0, 199)
    oh = (degi[:, None] ==
          lax.broadcasted_iota(jnp.int32, (BN, 200), 1)).astype(_f32)
    xb = jnp.maximum(_dot(a_ref[...], wa_ref[...]) + ba_ref[...], 0.0)
    xb = xb + _dot(oh, emb_ref[...], precision=lax.Precision.HIGHEST)
    x_ref[...] = xb
    c_ref[...] = _dot(xb, wit_ref[...])


def _tc_node_prologue(a, degp, wa, ba, emb, wit):
    return pl.pallas_call(
        _tc_node_prologue_body,
        grid=(GN,),
        in_specs=[
            pl.BlockSpec((BN, 70), lambda i: (i, 0)),
            pl.BlockSpec((BN, NW), lambda i: (i, 0)),
            pl.BlockSpec((70, H), lambda i: (0, 0)),
            pl.BlockSpec((1, H), lambda i: (0, 0)),
            pl.BlockSpec((200, H), lambda i: (0, 0)),
            pl.BlockSpec((H, H), lambda i: (0, 0)),
        ],
        out_specs=[
            pl.BlockSpec((BN, H), lambda i: (i, 0)),
            pl.BlockSpec((BN, H), lambda i: (i, 0)),
        ],
        out_shape=[
            jax.ShapeDtypeStruct((N, H), _f32),
            jax.ShapeDtypeStruct((N, H), _f32),
        ],
        compiler_params=pltpu.CompilerParams(
            dimension_semantics=("parallel",)),
    )(a, degp, wa, ba, emb, wit)


def _tc_mid_body(aggp_ref, wh_ref, b_ref):
    b_ref[...] = _dot(aggp_ref[0] + aggp_ref[1], wh_ref[...])


def _tc_mid(aggp, wh):
    return pl.pallas_call(
        _tc_mid_body,
        grid=(GN,),
        in_specs=[
            pl.BlockSpec((NC, BN, H), lambda i: (0, i, 0)),
            pl.BlockSpec((H, H), lambda i: (0, 0)),
        ],
        out_specs=pl.BlockSpec((BN, H), lambda i: (i, 0)),
        out_shape=jax.ShapeDtypeStruct((N, H), _f32),
        compiler_params=pltpu.CompilerParams(
            dimension_semantics=("parallel",)),
    )(aggp, wh)


def _tc_epilogue_body(x_ref, aggp_ref, wot_ref, wob_ref, w1_ref, b1_ref,
                      w2_ref, b2_ref, w3_ref, b3_ref, w4_ref, b4_ref,
                      out_ref, acc_ref):
    i = pl.program_id(0)

    @pl.when(i == 0)
    def _():
        acc_ref[...] = jnp.zeros_like(acc_ref)

    hn = aggp_ref[0] + aggp_ref[1]
    hv = jnp.maximum(_dot(x_ref[...], wot_ref[...]) +
                     _dot(hn, wob_ref[...]), 0.0)
    acc_ref[...] += jnp.sum(hv, axis=0, keepdims=True)

    @pl.when(i == GN - 1)
    def _():
        g = acc_ref[...] * _f32(1.0 / N)
        z = jnp.maximum(_dot(g, w1_ref[...]) + b1_ref[...], 0.0)
        z = jnp.maximum(_dot(z, w2_ref[...]) + b2_ref[...], 0.0)
        z = jnp.maximum(_dot(z, w3_ref[...]) + b3_ref[...], 0.0)
        out_ref[...] = _dot(z, w4_ref[...]) + b4_ref[...]


def _tc_epilogue(x, aggp, wot, wob, w1, b1, w2, b2, w3, b3, w4, b4):
    full = lambda *shape: pl.BlockSpec(shape, lambda i: (0,) * len(shape))
    return pl.pallas_call(
        _tc_epilogue_body,
        grid=(GN,),
        in_specs=[
            pl.BlockSpec((BN, H), lambda i: (i, 0)),
            pl.BlockSpec((NC, BN, H), lambda i: (0, i, 0)),
            full(H, H), full(H, H),
            full(H, H // 2), full(1, H // 2),
            full(H // 2, H // 4), full(1, H // 4),
            full(H // 4, H // 8), full(1, H // 8),
            full(H // 8, 1), full(1, 1),
        ],
        out_specs=pl.BlockSpec((1, 1), lambda i: (0, 0)),
        out_shape=jax.ShapeDtypeStruct((1, 1), _f32),
        scratch_shapes=[pltpu.VMEM((1, H), _f32)],
        compiler_params=pltpu.CompilerParams(
            dimension_semantics=("arbitrary",)),
    )(x, aggp, wot, wob, w1, b1, w2, b2, w3, b3, w4, b4)


def kernel(atom_feature, atom_coordinate, edge_feature, attention_weight,
           edge_index, W_atom, b_atom, W_edge, b_edge, degree_emb, W_i, W_h,
           W_o, W1, b1, W2, b2, W3, b3, W4, b4):
    src2d = edge_index[0].reshape(NW, NB, KB)
    dst2d = edge_index[1].reshape(NW, NB, KB)
    cx = atom_coordinate[:, 0]
    cy = atom_coordinate[:, 1]
    cz = atom_coordinate[:, 2]
    zerosH = jnp.zeros((NPT, H), _f32)
    zerosN = jnp.zeros((N,), _f32)

    e2 = _tc_edge_mm(edge_feature, W_edge, b_edge.reshape(1, HE), W_i[H:])

    degp, s = _sc_pass0()(cx, cy, cz, src2d, dst2d, zerosN)
    degp = degp.reshape(NC * NS, N).T

    scale2 = _tc_edge_scale(
        attention_weight.reshape(E // H, H), s.reshape(E // H, H))
    scale = scale2.reshape(NW, NB, KB)

    x, c = _tc_node_prologue(
        atom_feature, degp, W_atom, b_atom.reshape(1, H), degree_emb,
        W_i[:H])

    aggp, h0 = _make_edge_pass(True)(e2, c, src2d, dst2d, scale, zerosH)
    for _ in range(RADIUS):
        b = _tc_mid(aggp.reshape(NC, N, H), W_h)
        aggp, = _make_edge_pass(False)(h0, b, src2d, dst2d, scale, zerosH)

    out = _tc_epilogue(
        x, aggp.reshape(NC, N, H), W_o[:H], W_o[H:], W1, b1.reshape(1, H // 2),
        W2, b2.reshape(1, H // 4), W3, b3.reshape(1, H // 8),
        W4, b4.reshape(1, 1))
    return out.reshape(1)
